# baseline (speedup 1.0000x reference)
import functools

import jax
import jax.numpy as jnp
from jax import lax
from jax.experimental import pallas as pl
from jax.experimental.pallas import tpu as pltpu
from jax.experimental.pallas import tpu_sc as plsc

TOTAL = 2 * 8192 * 2048
ROWS = 16384
COLS = 2048
NUM_WORKERS = 32
ROWS_PER_WORKER = ROWS // NUM_WORKERS
RCHUNK = 8
CHUNK = RCHUNK * COLS
NCHUNK = ROWS_PER_WORKER // RCHUNK
UNROLL = 10
NSEG = 34
SEG_PAD = 40
KNOT_PAD = 56


def _sc_body(v_hbm, knots_hbm, coef_hbm, out_hbm,
             knots_v, c0, c1, c2, c3, c4, c5,
             in0, in1, out0, out1,
             sem_in0, sem_in1, sem_out0, sem_out1):
    wid = lax.axis_index("s") * 2 + lax.axis_index("c")
    base = wid * ROWS_PER_WORKER

    pltpu.sync_copy(knots_hbm, knots_v)
    coefs = (c0, c1, c2, c3, c4, c5)
    for j in range(6):
        pltpu.sync_copy(coef_hbm.at[j], coefs[j])

    inbufs = (in0, in1)
    outbufs = (out0, out1)
    sin = (sem_in0, sem_in1)
    sout = (sem_out0, sem_out1)

    def in_slice(g):
        return v_hbm.at[pl.ds(base + g * RCHUNK, RCHUNK), :]

    def out_slice(g):
        return out_hbm.at[pl.ds(base + g * RCHUNK, RCHUNK), :]

    pltpu.async_copy(in_slice(0), in0, sem_in0)
    pltpu.async_copy(in_slice(1), in1, sem_in1)

    def compute_chunk(ib, ob):
        @plsc.parallel_loop(0, CHUNK, 16, unroll=UNROLL)
        def _(off):
            r = lax.shift_right_logical(off, 11)
            c = lax.bitwise_and(off, COLS - 1)
            v = ib[r, pl.ds(c, 16)]
            lo = jnp.zeros((16,), jnp.int32)
            for b in (32, 16, 8, 4, 2, 1):
                hi = lo + b
                probe = plsc.load_gather(knots_v, [hi])
                lo = jnp.where(probe <= v, hi, lo)
            n2 = plsc.load_gather(c0, [lo])
            n1 = plsc.load_gather(c1, [lo])
            n0 = plsc.load_gather(c2, [lo])
            q2 = plsc.load_gather(c3, [lo])
            q1 = plsc.load_gather(c4, [lo])
            q0 = plsc.load_gather(c5, [lo])
            num = (n2 * v + n1) * v + n0
            den = (q2 * v + q1) * v + q0
            ob[r, pl.ds(c, 16)] = num / den

    def gbody(g2, carry):
        for b in range(2):
            g = g2 * 2 + b
            pltpu.make_async_copy(in_slice(g), inbufs[b], sin[b]).wait()
            @pl.when(g2 >= 1)
            def _():
                pltpu.make_async_copy(outbufs[b], out_slice(g - 2), sout[b]).wait()
            compute_chunk(inbufs[b], outbufs[b])
            pltpu.async_copy(outbufs[b], out_slice(g), sout[b])
            @pl.when(g2 < NCHUNK // 2 - 1)
            def _():
                pltpu.async_copy(in_slice(g + 2), inbufs[b], sin[b])
        return carry

    lax.fori_loop(0, NCHUNK // 2, gbody, 0)

    pltpu.make_async_copy(out0, out_slice(NCHUNK - 2), sem_out0).wait()
    pltpu.make_async_copy(out1, out_slice(NCHUNK - 1), sem_out1).wait()


@jax.jit
def _sc_spline(v_flat, knots_pad, coef):
    mesh = plsc.VectorSubcoreMesh(core_axis_name="c", subcore_axis_name="s")
    f = functools.partial(
        pl.kernel,
        out_type=jax.ShapeDtypeStruct((ROWS, COLS), jnp.float32),
        mesh=mesh,
        scratch_types=[
            pltpu.VMEM((KNOT_PAD,), jnp.float32),
            pltpu.VMEM((SEG_PAD,), jnp.float32),
            pltpu.VMEM((SEG_PAD,), jnp.float32),
            pltpu.VMEM((SEG_PAD,), jnp.float32),
            pltpu.VMEM((SEG_PAD,), jnp.float32),
            pltpu.VMEM((SEG_PAD,), jnp.float32),
            pltpu.VMEM((SEG_PAD,), jnp.float32),
            pltpu.VMEM((RCHUNK, COLS), jnp.float32),
            pltpu.VMEM((RCHUNK, COLS), jnp.float32),
            pltpu.VMEM((RCHUNK, COLS), jnp.float32),
            pltpu.VMEM((RCHUNK, COLS), jnp.float32),
            pltpu.SemaphoreType.DMA,
            pltpu.SemaphoreType.DMA,
            pltpu.SemaphoreType.DMA,
            pltpu.SemaphoreType.DMA,
        ],
        compiler_params=pltpu.CompilerParams(needs_layout_passes=False),
    )(_sc_body)
    return f(v_flat, knots_pad, coef)


def kernel(input_data, x_pos, x_neg, y_pos, y_neg, ln_d):
    n = y_pos.shape[-1]
    wp = jnp.exp(x_pos).reshape(1, n, 2).sum(-1)
    wn = jnp.exp(x_neg).reshape(1, n, 2).sum(-1)
    hp = jnp.exp(y_pos)
    hn = jnp.exp(y_neg)
    widths = jnp.concatenate([wn[:, ::-1], wp], axis=-1)
    heights = jnp.concatenate([hn[:, ::-1], hp], axis=-1)
    cw = jnp.cumsum(widths, axis=-1)
    X = jnp.concatenate([jnp.zeros_like(cw[:, :1]), cw], axis=-1)
    X = (X - X[:, n:n + 1])[0]
    ch = jnp.cumsum(heights, axis=-1)
    Y = jnp.concatenate([jnp.zeros_like(ch[:, :1]), ch], axis=-1)
    Y = (Y - Y[:, n:n + 1])[0]
    D = jnp.exp(ln_d)[0]

    xk = X[:-1]
    yk = Y[:-1]
    dk = D[:-1]
    dk1 = D[1:]
    w = X[1:] - xk
    h = Y[1:] - yk
    s = h / w
    dd = dk + dk1 - 2.0 * s
    aN = h * s - h * dk - yk * dd
    bN = w * (yk * dd + h * dk)
    cN = yk * s * w * w
    aQ = -dd
    bQ = dd * w
    cQ = s * w * w

    def shift(a, b, c):
        return a, b - 2.0 * a * xk, (a * xk - b) * xk + c

    n2, n1, n0 = shift(aN, bN, cN)
    q2, q1, q0 = shift(aQ, bQ, cQ)

    zero1 = jnp.zeros((1,), jnp.float32)
    one1 = jnp.ones((1,), jnp.float32)
    padz = jnp.zeros((SEG_PAD - NSEG,), jnp.float32)
    lt_n1 = D[:1]
    lt_n0 = (Y[0] - D[0] * X[0])[None]
    rt_n1 = D[-1:]
    rt_n0 = (Y[-1] - D[-1] * X[-1])[None]
    en2 = jnp.concatenate([zero1, n2, zero1, padz])
    en1 = jnp.concatenate([lt_n1, n1, rt_n1, padz])
    en0 = jnp.concatenate([lt_n0, n0, rt_n0, padz])
    eq2 = jnp.concatenate([zero1, q2, zero1, padz])
    eq1 = jnp.concatenate([zero1, q1, zero1, padz])
    eq0 = jnp.concatenate([one1, q0, one1, padz])
    coef = jnp.stack([en2, en1, en0, eq2, eq1, eq0])

    big = jnp.full((KNOT_PAD - NSEG,), jnp.inf, jnp.float32)
    knots_pad = jnp.concatenate([X[:1], X, big])

    v = input_data.reshape(ROWS, COLS)
    out = _sc_spline(v, knots_pad, coef)
    return out.reshape(input_data.shape)


# speedup vs baseline: 1.4634x; 1.4634x over previous
import functools

import jax
import jax.numpy as jnp
from jax import lax
from jax.experimental import pallas as pl
from jax.experimental.pallas import tpu as pltpu
from jax.experimental.pallas import tpu_sc as plsc

TOTAL = 2 * 8192 * 2048
ROWS = 16384
COLS = 2048
NUM_WORKERS = 32
ROWS_PER_WORKER = ROWS // NUM_WORKERS
RCHUNK = 8
CHUNK = RCHUNK * COLS
NCHUNK = ROWS_PER_WORKER // RCHUNK
UNROLL = 8
NSEG = 34
SEG_PAD = 40
KNOT_PAD = 56


def _sc_body(v_hbm, knots_hbm, coef_hbm, out_hbm,
             knots_v, c0, c1, c2, c3, c4, c5,
             in0, in1, out0, out1,
             sem_in0, sem_in1, sem_out0, sem_out1):
    wid = lax.axis_index("s") * 2 + lax.axis_index("c")
    base = wid * ROWS_PER_WORKER

    pltpu.sync_copy(knots_hbm, knots_v)
    coefs = (c0, c1, c2, c3, c4, c5)
    for j in range(6):
        pltpu.sync_copy(coef_hbm.at[j], coefs[j])

    inbufs = (in0, in1)
    outbufs = (out0, out1)
    sin = (sem_in0, sem_in1)
    sout = (sem_out0, sem_out1)

    def in_slice(g):
        return v_hbm.at[pl.ds(base + g * RCHUNK, RCHUNK), :]

    def out_slice(g):
        return out_hbm.at[pl.ds(base + g * RCHUNK, RCHUNK), :]

    pltpu.async_copy(in_slice(0), in0, sem_in0)
    pltpu.async_copy(in_slice(1), in1, sem_in1)

    def compute_chunk(ib, ob):
        @plsc.parallel_loop(0, CHUNK, 16, unroll=UNROLL)
        def _(off):
            r = lax.shift_right_logical(off, 11)
            c = lax.bitwise_and(off, COLS - 1)
            v = ib[r, pl.ds(c, 16)]
            lo = jnp.zeros((16,), jnp.int32)
            for b in (32, 16, 8, 4, 2, 1):
                hi = lo + b
                probe = plsc.load_gather(knots_v, [hi])
                lo = jnp.where(probe <= v, hi, lo)
            n2 = plsc.load_gather(c0, [lo])
            n1 = plsc.load_gather(c1, [lo])
            n0 = plsc.load_gather(c2, [lo])
            q2 = plsc.load_gather(c3, [lo])
            q1 = plsc.load_gather(c4, [lo])
            q0 = plsc.load_gather(c5, [lo])
            num = (n2 * v + n1) * v + n0
            den = (q2 * v + q1) * v + q0
            ob[r, pl.ds(c, 16)] = num / den

    def gbody(g2, carry):
        for b in range(2):
            g = g2 * 2 + b
            pltpu.make_async_copy(in_slice(g), inbufs[b], sin[b]).wait()
            @pl.when(g2 >= 1)
            def _():
                pltpu.make_async_copy(outbufs[b], out_slice(g - 2), sout[b]).wait()
            compute_chunk(inbufs[b], outbufs[b])
            pltpu.async_copy(outbufs[b], out_slice(g), sout[b])
            @pl.when(g2 < NCHUNK // 2 - 1)
            def _():
                pltpu.async_copy(in_slice(g + 2), inbufs[b], sin[b])
        return carry

    lax.fori_loop(0, NCHUNK // 2, gbody, 0)

    pltpu.make_async_copy(out0, out_slice(NCHUNK - 2), sem_out0).wait()
    pltpu.make_async_copy(out1, out_slice(NCHUNK - 1), sem_out1).wait()


@jax.jit
def _sc_spline(v_flat, knots_pad, coef):
    mesh = plsc.VectorSubcoreMesh(core_axis_name="c", subcore_axis_name="s")
    f = functools.partial(
        pl.kernel,
        out_type=jax.ShapeDtypeStruct((ROWS, COLS), jnp.float32),
        mesh=mesh,
        scratch_types=[
            pltpu.VMEM((KNOT_PAD,), jnp.float32),
            pltpu.VMEM((SEG_PAD,), jnp.float32),
            pltpu.VMEM((SEG_PAD,), jnp.float32),
            pltpu.VMEM((SEG_PAD,), jnp.float32),
            pltpu.VMEM((SEG_PAD,), jnp.float32),
            pltpu.VMEM((SEG_PAD,), jnp.float32),
            pltpu.VMEM((SEG_PAD,), jnp.float32),
            pltpu.VMEM((RCHUNK, COLS), jnp.float32),
            pltpu.VMEM((RCHUNK, COLS), jnp.float32),
            pltpu.VMEM((RCHUNK, COLS), jnp.float32),
            pltpu.VMEM((RCHUNK, COLS), jnp.float32),
            pltpu.SemaphoreType.DMA,
            pltpu.SemaphoreType.DMA,
            pltpu.SemaphoreType.DMA,
            pltpu.SemaphoreType.DMA,
        ],
        compiler_params=pltpu.CompilerParams(needs_layout_passes=False),
    )(_sc_body)
    return f(v_flat, knots_pad, coef)


def kernel(input_data, x_pos, x_neg, y_pos, y_neg, ln_d):
    n = y_pos.shape[-1]
    wp = jnp.exp(x_pos).reshape(1, n, 2).sum(-1)
    wn = jnp.exp(x_neg).reshape(1, n, 2).sum(-1)
    hp = jnp.exp(y_pos)
    hn = jnp.exp(y_neg)
    widths = jnp.concatenate([wn[:, ::-1], wp], axis=-1)
    heights = jnp.concatenate([hn[:, ::-1], hp], axis=-1)
    cw = jnp.cumsum(widths, axis=-1)
    X = jnp.concatenate([jnp.zeros_like(cw[:, :1]), cw], axis=-1)
    X = (X - X[:, n:n + 1])[0]
    ch = jnp.cumsum(heights, axis=-1)
    Y = jnp.concatenate([jnp.zeros_like(ch[:, :1]), ch], axis=-1)
    Y = (Y - Y[:, n:n + 1])[0]
    D = jnp.exp(ln_d)[0]

    xk = X[:-1]
    yk = Y[:-1]
    dk = D[:-1]
    dk1 = D[1:]
    w = X[1:] - xk
    h = Y[1:] - yk
    s = h / w
    dd = dk + dk1 - 2.0 * s
    aN = h * s - h * dk - yk * dd
    bN = w * (yk * dd + h * dk)
    cN = yk * s * w * w
    aQ = -dd
    bQ = dd * w
    cQ = s * w * w

    def shift(a, b, c):
        return a, b - 2.0 * a * xk, (a * xk - b) * xk + c

    n2, n1, n0 = shift(aN, bN, cN)
    q2, q1, q0 = shift(aQ, bQ, cQ)

    zero1 = jnp.zeros((1,), jnp.float32)
    one1 = jnp.ones((1,), jnp.float32)
    padz = jnp.zeros((SEG_PAD - NSEG,), jnp.float32)
    lt_n1 = D[:1]
    lt_n0 = (Y[0] - D[0] * X[0])[None]
    rt_n1 = D[-1:]
    rt_n0 = (Y[-1] - D[-1] * X[-1])[None]
    en2 = jnp.concatenate([zero1, n2, zero1, padz])
    en1 = jnp.concatenate([lt_n1, n1, rt_n1, padz])
    en0 = jnp.concatenate([lt_n0, n0, rt_n0, padz])
    eq2 = jnp.concatenate([zero1, q2, zero1, padz])
    eq1 = jnp.concatenate([zero1, q1, zero1, padz])
    eq0 = jnp.concatenate([one1, q0, one1, padz])
    coef = jnp.stack([en2, en1, en0, eq2, eq1, eq0])

    big = jnp.full((KNOT_PAD - NSEG,), jnp.inf, jnp.float32)
    knots_pad = jnp.concatenate([X[:1], X, big])

    v = input_data.reshape(ROWS, COLS)
    out = _sc_spline(v, knots_pad, coef)
    return out.reshape(input_data.shape)
